# baseline (speedup 1.0000x reference)
import functools

import jax
import jax.numpy as jnp
from jax import lax
from jax.experimental import pallas as pl
from jax.experimental.pallas import tpu as pltpu
from jax.experimental.pallas import tpu_sc as plsc

B, V, C = 4, 40962, 256
M = (V + 6) // 4
NW = 32
NPW = 352
NB = 16
NCH = NPW // NB
ROWS = NB * 7
IDXW = NPW * 7
IDX_PAD = NW * IDXW
UN = 4
INV7 = float(1.0 / 7.0)
PART_NS = (M // NB) * NB
PART_ROWS = M - PART_NS


def _pool_body(x_hbm, no_hbm, out_hbm, idx_raw, gbuf, obuf,
               sem_g0, sem_g1, sem_w0, sem_w1):
    wid = lax.axis_index("s") * 2 + lax.axis_index("c")
    node_base = wid * NPW

    pltpu.sync_copy(no_hbm.at[pl.ds(wid * IDXW, IDXW)], idx_raw)

    lane7 = 7 * lax.iota(jnp.int32, 16)
    sem_g = (sem_g0, sem_g1)
    sem_w = (sem_w0, sem_w1)

    def ns_of(c):
        return node_base + c * NB

    def g_desc(b, c, par):
        idxsl = idx_raw.at[pl.ds(c * ROWS, ROWS)]
        return pltpu.make_async_copy(x_hbm.at[b].at[idxsl],
                                     gbuf.at[par].at[pl.ds(0, ROWS)],
                                     sem_g[par])

    def w_full_desc(b, c, par):
        return pltpu.make_async_copy(obuf.at[par],
                                     out_hbm.at[b].at[pl.ds(ns_of(c), NB)],
                                     sem_w[par])

    def w_part_desc(b, par):
        return pltpu.make_async_copy(obuf.at[par].at[pl.ds(0, PART_ROWS)],
                                     out_hbm.at[b].at[pl.ds(PART_NS, PART_ROWS)],
                                     sem_w[par])

    def w_act(b, c, par, act):
        ns = ns_of(c)

        @pl.when(ns <= M - NB)
        def _():
            act(w_full_desc(b, c, par))

        @pl.when(ns == PART_NS)
        def _():
            act(w_part_desc(b, par))

    def g_act(b, c, par, act):
        @pl.when(ns_of(c) < M)
        def _():
            act(g_desc(b, c, par))

    def compute_chunk(par):
        for oc in range(16):
            t = [lane7 + (112 * oc + k) for k in range(7)]
            rows = [lax.shift_right_logical(tk, 8) for tk in t]
            cols = [lax.bitwise_and(tk, 255) for tk in t]

            def nbody(ni, _):
                for u in range(UN):
                    n = ni * UN + u
                    roff = 7 * n
                    g = [plsc.load_gather(gbuf.at[par],
                                          [rows[k] + roff, cols[k]])
                         for k in range(7)]
                    acc = ((g[0] + g[1]) + (g[2] + g[3])) + \
                          ((g[4] + g[5]) + g[6])
                    obuf[par, n, pl.ds(oc * 16, 16)] = acc * INV7
                return 0
            lax.fori_loop(0, NB // UN, nbody, 0)

    def batch_body(b, _):
        g_act(b, 0, 0, lambda d: d.start())
        g_act(b, 1, 1, lambda d: d.start())

        def pair_body(i, _):
            for par in (0, 1):
                c = 2 * i + par
                g_act(b, c, par, lambda d: d.wait())

                @pl.when(i >= 1)
                def _():
                    w_act(b, c - 2, par, lambda d: d.wait())

                @pl.when(ns_of(c) < M)
                def _():
                    compute_chunk(par)

                w_act(b, c, par, lambda d: d.start())

                @pl.when(i < NCH // 2 - 1)
                def _():
                    g_act(b, c + 2, par, lambda d: d.start())
            return 0
        lax.fori_loop(0, NCH // 2, pair_body, 0)

        w_act(b, NCH - 2, 0, lambda d: d.wait())
        w_act(b, NCH - 1, 1, lambda d: d.wait())
        return 0
    lax.fori_loop(0, B, batch_body, 0)


@jax.jit
def _pool(x, no_pad):
    mesh = plsc.VectorSubcoreMesh(core_axis_name="c", subcore_axis_name="s")
    f = pl.kernel(
        _pool_body,
        out_type=jax.ShapeDtypeStruct((B, M, C), jnp.float32),
        mesh=mesh,
        compiler_params=pltpu.CompilerParams(
            use_tc_tiling_on_sc=True, needs_layout_passes=False),
        scratch_types=[
            pltpu.VMEM((IDXW,), jnp.int32),
            pltpu.VMEM((2, ROWS + 1, C), jnp.float32),
            pltpu.VMEM((2, NB, C), jnp.float32),
            pltpu.SemaphoreType.DMA,
            pltpu.SemaphoreType.DMA,
            pltpu.SemaphoreType.DMA,
            pltpu.SemaphoreType.DMA,
        ],
    )
    return f(x, no_pad)


def kernel(x, neigh_orders):
    no_pad = jnp.pad(neigh_orders[: M * 7], (0, IDX_PAD - M * 7))
    return _pool(x, no_pad)


# speedup vs baseline: 1.3072x; 1.3072x over previous
import functools

import jax
import jax.numpy as jnp
from jax import lax
from jax.experimental import pallas as pl
from jax.experimental.pallas import tpu as pltpu
from jax.experimental.pallas import tpu_sc as plsc

B, V, C = 4, 40962, 256
M = (V + 6) // 4
NW = 32
NPW = 352
NB = 8
NCH = NPW // NB
ROWS = NB * 7
EIDX = ROWS * 8
IDXW = NPW * 7
IDX_PAD = NW * IDXW
UN = 2
INV7 = float(1.0 / 7.0)
PART_NS = (M // NB) * NB
PART_ROWS = M - PART_NS


def _pool_body(x_hbm, eno_hbm, out_hbm, ebuf0, ebuf1, gbuf, obuf,
               sem_g0, sem_g1, sem_w0, sem_i0, sem_i1):
    wid = lax.axis_index("s") * 2 + lax.axis_index("c")
    node_base = wid * NPW

    lane7 = 7 * lax.iota(jnp.int32, 16)
    sem_g = (sem_g0, sem_g1)
    sem_i = (sem_i0, sem_i1)
    ebuf = (ebuf0, ebuf1)

    def ns_of(c):
        return node_base + c * NB

    def i_desc(c, par):
        return pltpu.make_async_copy(
            eno_hbm.at[pl.ds((wid * IDXW + c * ROWS) * 8, 512)],
            ebuf[par], sem_i[par])

    G_SPLIT = ((0, 128), (128, 128), (256, 128), (384, 64))

    def g_desc(c, par, j):
        off, sz = G_SPLIT[j]
        return pltpu.make_async_copy(
            x_hbm.at[ebuf[par].at[pl.ds(off, sz)]],
            gbuf.at[par].at[pl.ds(off, sz)],
            sem_g[par])

    def w_full_desc(c):
        return pltpu.make_async_copy(obuf,
                                     out_hbm.at[pl.ds(ns_of(c), NB)],
                                     sem_w0)

    def w_part_desc():
        return pltpu.make_async_copy(obuf.at[pl.ds(0, PART_ROWS)],
                                     out_hbm.at[pl.ds(PART_NS, PART_ROWS)],
                                     sem_w0)

    def w_act(c, act):
        ns = ns_of(c)

        @pl.when(ns <= M - NB)
        def _():
            act(w_full_desc(c))

        @pl.when(ns == PART_NS)
        def _():
            act(w_part_desc())

    def i_act(c, par, act):
        @pl.when(ns_of(c) < M)
        def _():
            act(i_desc(c, par))

    def g_act(c, par, act):
        @pl.when(ns_of(c) < M)
        def _():
            for j in range(4):
                act(g_desc(c, par, j))

    def by_par(c, act_par):
        lo = lax.rem(c, 2)

        @pl.when(lo == 0)
        def _():
            act_par(0)

        @pl.when(lo == 1)
        def _():
            act_par(1)

    def compute_chunk(par):
        for b in range(B):
            for oc in range(16):
                t = [lane7 + (112 * oc + k) for k in range(7)]
                rows = [(lax.shift_right_logical(tk, 8) * 8
                         + lax.bitwise_and(lax.shift_right_logical(tk, 7), 1)
                         * 4 + b) for tk in t]
                cols = [lax.bitwise_and(tk, 127) for tk in t]

                def nbody(ni, _):
                    for u in range(UN):
                        n = ni * UN + u
                        roff = 56 * n
                        g = [plsc.load_gather(gbuf.at[par],
                                              [rows[k] + roff, cols[k]])
                             for k in range(7)]
                        acc = ((g[0] + g[1]) + (g[2] + g[3])) + \
                              ((g[4] + g[5]) + g[6])
                        obuf[n, (oc // 8) * 4 + b,
                             pl.ds((oc % 8) * 16, 16)] = acc * INV7
                    return 0
                lax.fori_loop(0, NB // UN, nbody, 0)

    i_act(0, 0, lambda d: d.start())
    i_act(1, 1, lambda d: d.start())
    i_act(0, 0, lambda d: d.wait())
    g_act(0, 0, lambda d: d.start())
    i_act(1, 1, lambda d: d.wait())
    g_act(1, 1, lambda d: d.start())

    def chunk_body(c, _):
        by_par(c, lambda par: g_act(c, par, lambda d: d.wait()))

        @pl.when(c + 2 < NCH)
        def _():
            by_par(c, lambda par: i_act(c + 2, par, lambda d: d.start()))

        @pl.when(c >= 1)
        def _():
            w_act(c - 1, lambda d: d.wait())

        @pl.when(ns_of(c) < M)
        def _():
            compute_chunk(lax.rem(c, 2))

        w_act(c, lambda d: d.start())

        @pl.when(c + 2 < NCH)
        def _():
            by_par(c, lambda par: i_act(c + 2, par, lambda d: d.wait()))
            by_par(c, lambda par: g_act(c + 2, par, lambda d: d.start()))
        return 0
    lax.fori_loop(0, NCH, chunk_body, 0)

    w_act(NCH - 1, lambda d: d.wait())


@jax.jit
def _pool(xt, eno):
    mesh = plsc.VectorSubcoreMesh(core_axis_name="c", subcore_axis_name="s")
    f = pl.kernel(
        _pool_body,
        out_type=jax.ShapeDtypeStruct((M, 8, 128), jnp.float32),
        mesh=mesh,
        compiler_params=pltpu.CompilerParams(
            use_tc_tiling_on_sc=True, needs_layout_passes=False),
        scratch_types=[
            pltpu.VMEM((512,), jnp.int32),
            pltpu.VMEM((512,), jnp.int32),
            pltpu.VMEM((2, EIDX, 128), jnp.float32),
            pltpu.VMEM((NB, 8, 128), jnp.float32),
            pltpu.SemaphoreType.DMA,
            pltpu.SemaphoreType.DMA,
            pltpu.SemaphoreType.DMA,
            pltpu.SemaphoreType.DMA,
            pltpu.SemaphoreType.DMA,
        ],
    )
    return f(xt, eno)


def kernel(x, neigh_orders):
    xt = x.reshape(B, V, 2, 128).transpose(1, 2, 0, 3).reshape(V * 8, 128)
    no_pad = jnp.pad(neigh_orders[: M * 7], (0, IDX_PAD - M * 7 + 64))
    eno = (no_pad[:, None] * 8 + jnp.arange(8, dtype=jnp.int32)).reshape(-1)
    ot = _pool(xt, eno)
    return ot.reshape(M, 2, B, 128).transpose(2, 0, 1, 3).reshape(B, M, C)
